# baseline (speedup 1.0000x reference)
import functools

import jax
import jax.numpy as jnp
from jax import lax
from jax.experimental import pallas as pl
from jax.experimental.pallas import tpu as pltpu
from jax.experimental.pallas import tpu_sc as plsc

B = 16384
D = 32
L = 16
NC = 2
NS = 16
NW = NC * NS
BPW = B // NW


@functools.partial(
    pl.kernel,
    out_type=jax.ShapeDtypeStruct((B,), jnp.float32),
    mesh=plsc.VectorSubcoreMesh(core_axis_name="c", subcore_axis_name="s",
                                num_cores=NC, num_subcores=NS),
    compiler_params=pltpu.CompilerParams(needs_layout_passes=False,
                                         use_tc_tiling_on_sc=False),
    scratch_types=[
        pltpu.VMEM((BPW,), jnp.int32),
        pltpu.VMEM((BPW,), jnp.int32),
        pltpu.VMEM((BPW, D), jnp.float32),
        pltpu.VMEM((BPW, D), jnp.float32),
        pltpu.VMEM((BPW,), jnp.float32),
        pltpu.VMEM((BPW,), jnp.float32),
        pltpu.VMEM((L,), jnp.float32),
        pltpu.VMEM((BPW,), jnp.float32),
        pltpu.SemaphoreType.DMA,
        pltpu.SemaphoreType.DMA,
        pltpu.SemaphoreType.DMA,
        pltpu.SemaphoreType.DMA,
    ],
)
def _mf_kernel(uid_hbm, iid_hbm, ut_hbm, it_hbm, ub_hbm, ib_hbm, gb_hbm,
               out_hbm, uid_v, iid_v, urows_v, irows_v, ub_v, ib_v, gb_v,
               out_v, sem_u, sem_i, sem_ub, sem_ib):
    wid = lax.axis_index("s") * NC + lax.axis_index("c")
    base = wid * BPW
    pltpu.sync_copy(uid_hbm.at[pl.ds(base, BPW)], uid_v)
    pltpu.sync_copy(iid_hbm.at[pl.ds(base, BPW)], iid_v)
    pltpu.sync_copy(gb_hbm, gb_v)
    cu = pltpu.async_copy(ut_hbm.at[uid_v], urows_v, sem_u)
    ci = pltpu.async_copy(it_hbm.at[iid_v], irows_v, sem_i)
    cub = pltpu.async_copy(ub_hbm.at[uid_v], ub_v, sem_ub)
    cib = pltpu.async_copy(ib_hbm.at[iid_v], ib_v, sem_ib)
    cu.wait()
    ci.wait()
    cub.wait()
    cib.wait()
    gb = gb_v[...]

    def body(g, carry):
        row = g * L + lax.iota(jnp.int32, L)
        acc = ub_v[pl.ds(g * L, L)] + ib_v[pl.ds(g * L, L)] + gb
        for d in range(D):
            col = jnp.full((L,), d, jnp.int32)
            u = plsc.load_gather(urows_v, [row, col])
            it = plsc.load_gather(irows_v, [row, col])
            acc = acc + u * it
        out_v[pl.ds(g * L, L)] = acc
        return carry

    lax.fori_loop(0, BPW // L, body, 0)
    pltpu.sync_copy(out_v, out_hbm.at[pl.ds(base, BPW)])


def kernel(user_ids, item_ids, user_table, item_table, user_bias_table,
           item_bias_table, global_bias):
    gb16 = jnp.broadcast_to(global_bias.astype(jnp.float32), (L,))
    return _mf_kernel(user_ids, item_ids, user_table, item_table,
                      user_bias_table.reshape(-1), item_bias_table.reshape(-1),
                      gb16)


# speedup vs baseline: 2.6200x; 2.6200x over previous
import functools

import jax
import jax.numpy as jnp
from jax import lax
from jax.experimental import pallas as pl
from jax.experimental.pallas import tpu as pltpu
from jax.experimental.pallas import tpu_sc as plsc

B = 16384
D = 32
L = 16
NC = 2
NS = 16
NW = NC * NS
V = 1000000
G = 1024
MAIN = 999936
NFULL = 976
TAIL = V - MAIN

_SC_MESH = plsc.VectorSubcoreMesh(core_axis_name="c", subcore_axis_name="s",
                                  num_cores=NC, num_subcores=NS)
_PARAMS = pltpu.CompilerParams(needs_layout_passes=False)


def _iota():
    return lax.iota(jnp.int32, L)


def _bc(x, dtype=jnp.float32):
    return jnp.full((L,), x, dtype)


def _match_pass(ids_v, wid, cnt_init, matchb_v, matchid_v):
    def body(k, cnt):
        idv = ids_v[pl.ds(k * L, L)]
        bv = k * L + _iota()
        chunk = lax.shift_right_logical(idv, 10)
        m = jnp.logical_and(jnp.bitwise_and(chunk, NW - 1) == wid,
                            idv < MAIN)
        m = jnp.logical_or(
            m, jnp.logical_and(idv >= MAIN, _bc(wid, jnp.int32) == NW - 1))
        minc = m.astype(jnp.int32)
        pos = cnt + plsc.cumsum(minc) - 1
        plsc.store_scatter(matchid_v, [pos], idv, mask=m)
        plsc.store_scatter(matchb_v, [pos], bv, mask=m)
        return cnt + plsc.all_reduce_population_count(m)[0]
    return lax.fori_loop(0, B // L, body, cnt_init)


def _chunk_select(c, lo, width, cnt, matchb_v, matchid_v, minib_v, minioff_v):
    def body(k, mc):
        lane = k * L + _iota()
        idv = matchid_v[pl.ds(k * L, L)]
        bv = matchb_v[pl.ds(k * L, L)]
        m = jnp.logical_and(lane < cnt,
                            jnp.logical_and(idv >= lo, idv < lo + width))
        pos = mc + plsc.cumsum(m.astype(jnp.int32)) - 1
        plsc.store_scatter(minioff_v, [pos], idv - lo, mask=m)
        plsc.store_scatter(minib_v, [pos], bv, mask=m)
        return mc + plsc.all_reduce_population_count(m)[0]
    return lax.fori_loop(0, (cnt + L - 1) // L, body, 0)


@functools.partial(
    pl.kernel,
    out_type=jax.ShapeDtypeStruct((B, 128), jnp.float32),
    mesh=_SC_MESH,
    compiler_params=_PARAMS,
    scratch_types=[
        pltpu.VMEM((B,), jnp.int32),
        pltpu.VMEM((B,), jnp.int32),
        pltpu.VMEM((B,), jnp.int32),
        pltpu.VMEM((B,), jnp.int32),
        pltpu.VMEM((B,), jnp.int32),
        pltpu.VMEM((D, G), jnp.float32),
        pltpu.VMEM((L, 128), jnp.float32),
        pltpu.VMEM((L,), jnp.int32),
        pltpu.SemaphoreType.DMA,
    ],
)
def _k1(uid_hbm, ut_hbm, u_rows_hbm, uids_v, matchb_v, matchid_v,
        minib_v, minioff_v, slab_v, rows_v, bidx_v, sem):
    wid = lax.axis_index("s") * NC + lax.axis_index("c")
    pltpu.sync_copy(uid_hbm, uids_v)
    cnt = _match_pass(uids_v, wid, 0, matchb_v, matchid_v)

    dlo = _iota()
    dhi = dlo + L

    def emit_groups(mc, width):
        def gbody(g, carry):
            lane = g * L + _iota()
            valid = lane < mc
            offv = jnp.where(valid, minioff_v[pl.ds(g * L, L)], 0)
            bv = jnp.where(valid, minib_v[pl.ds(g * L, L)], -1)
            bidx_v[...] = bv
            for l in range(L):
                off = _bc(offv[l], jnp.int32)
                rows_v[l, pl.ds(0, L)] = plsc.load_gather(slab_v, [dlo, off])
                rows_v[l, pl.ds(L, L)] = plsc.load_gather(slab_v, [dhi, off])
            pltpu.async_copy(
                rows_v, u_rows_hbm.at[plsc.Indices(bidx_v, ignored_value=-1)],
                sem).wait()
            return carry
        lax.fori_loop(0, (mc + L - 1) // L, gbody, 0)

    def chunk_body(j, carry):
        c = wid + j * NW

        @pl.when(c < NFULL)
        def _():
            lo = c * G
            pltpu.sync_copy(ut_hbm.at[:, pl.ds(pl.multiple_of(lo, 128), G)],
                            slab_v)
            mc = _chunk_select(c, lo, G, cnt, matchb_v, matchid_v,
                               minib_v, minioff_v)
            emit_groups(mc, G)
        return carry

    lax.fori_loop(0, NFULL // NW + 1, chunk_body, 0)

    @pl.when(wid == L)
    def _():
        lo = NFULL * G
        pltpu.sync_copy(ut_hbm.at[:, pl.ds(lo, 512)],
                        slab_v.at[:, pl.ds(0, 512)])
        mc = _chunk_select(NFULL, lo, 512, cnt, matchb_v, matchid_v,
                           minib_v, minioff_v)
        emit_groups(mc, 512)

    @pl.when(wid == NW - 1)
    def _():
        for dd in range(D):
            pltpu.sync_copy(ut_hbm.at[dd, pl.ds(MAIN, TAIL)],
                            slab_v.at[dd, pl.ds(0, TAIL)])
        mc = _chunk_select(0, MAIN, TAIL, cnt, matchb_v, matchid_v,
                           minib_v, minioff_v)
        emit_groups(mc, TAIL)


@functools.partial(
    pl.kernel,
    out_type=jax.ShapeDtypeStruct((B,), jnp.float32),
    mesh=_SC_MESH,
    compiler_params=_PARAMS,
    scratch_types=[
        pltpu.VMEM((B,), jnp.int32),
        pltpu.VMEM((B,), jnp.int32),
        pltpu.VMEM((B,), jnp.int32),
        pltpu.VMEM((B,), jnp.int32),
        pltpu.VMEM((B,), jnp.int32),
        pltpu.VMEM((D, G), jnp.float32),
        pltpu.VMEM((L, 128), jnp.float32),
        pltpu.VMEM((L,), jnp.int32),
        pltpu.VMEM((L,), jnp.int32),
        pltpu.VMEM((L,), jnp.int32),
        pltpu.VMEM((L,), jnp.float32),
        pltpu.VMEM((L,), jnp.float32),
        pltpu.VMEM((L,), jnp.float32),
        pltpu.VMEM((L,), jnp.float32),
        pltpu.SemaphoreType.DMA,
        pltpu.SemaphoreType.DMA,
    ],
)
def _k2(iid_hbm, it_hbm, u_rows_hbm, uid_hbm, ub_hbm, ib_hbm, gb_hbm,
        out_hbm, iids_v, matchb_v, matchid_v, minib_v, minioff_v, slab_v,
        urows_v, bidx_v, uidg_v, iidg_v, ubv_v, ibv_v, gb_v, res_v,
        sem, sem2):
    wid = lax.axis_index("s") * NC + lax.axis_index("c")
    pltpu.sync_copy(iid_hbm, iids_v)
    pltpu.sync_copy(gb_hbm, gb_v)
    cnt = _match_pass(iids_v, wid, 0, matchb_v, matchid_v)

    dlo = _iota()
    dhi = dlo + L

    def emit_groups(mc, lo):
        def gbody(g, carry):
            lane = g * L + _iota()
            valid = lane < mc
            offv = jnp.where(valid, minioff_v[pl.ds(g * L, L)], 0)
            bv = jnp.where(valid, minib_v[pl.ds(g * L, L)], -1)
            bidx_v[...] = bv
            iidg_v[...] = jnp.where(valid, offv + lo, 0)
            cu = pltpu.async_copy(
                u_rows_hbm.at[plsc.Indices(bidx_v, ignored_value=-1)],
                urows_v, sem)
            cid = pltpu.async_copy(
                uid_hbm.at[plsc.Indices(bidx_v, ignored_value=-1)],
                uidg_v, sem2)
            cu.wait()
            cid.wait()
            pltpu.async_copy(ub_hbm.at[plsc.Indices(uidg_v)], ubv_v,
                             sem).wait()
            pltpu.async_copy(ib_hbm.at[plsc.Indices(iidg_v)], ibv_v,
                             sem2).wait()
            acc = jnp.zeros((L,), jnp.float32)
            for l in range(L):
                off = _bc(offv[l], jnp.int32)
                ilo = plsc.load_gather(slab_v, [dlo, off])
                ihi = plsc.load_gather(slab_v, [dhi, off])
                ulo = urows_v[l, pl.ds(0, L)]
                uhi = urows_v[l, pl.ds(L, L)]
                dot = jnp.sum(ulo * ilo + uhi * ihi)
                acc = jnp.where(_iota() == l, _bc(dot), acc)
            res_v[...] = acc + ubv_v[...] + ibv_v[...] + gb_v[...]
            pltpu.async_copy(
                res_v, out_hbm.at[plsc.Indices(bidx_v, ignored_value=-1)],
                sem).wait()
            return carry
        lax.fori_loop(0, (mc + L - 1) // L, gbody, 0)

    def chunk_body(j, carry):
        c = wid + j * NW

        @pl.when(c < NFULL)
        def _():
            lo = c * G
            pltpu.sync_copy(it_hbm.at[:, pl.ds(pl.multiple_of(lo, 128), G)],
                            slab_v)
            mc = _chunk_select(c, lo, G, cnt, matchb_v, matchid_v,
                               minib_v, minioff_v)
            emit_groups(mc, lo)
        return carry

    lax.fori_loop(0, NFULL // NW + 1, chunk_body, 0)

    @pl.when(wid == L)
    def _():
        lo = NFULL * G
        pltpu.sync_copy(it_hbm.at[:, pl.ds(lo, 512)],
                        slab_v.at[:, pl.ds(0, 512)])
        mc = _chunk_select(NFULL, lo, 512, cnt, matchb_v, matchid_v,
                           minib_v, minioff_v)
        emit_groups(mc, lo)

    @pl.when(wid == NW - 1)
    def _():
        for dd in range(D):
            pltpu.sync_copy(it_hbm.at[dd, pl.ds(MAIN, TAIL)],
                            slab_v.at[dd, pl.ds(0, TAIL)])
        mc = _chunk_select(0, MAIN, TAIL, cnt, matchb_v, matchid_v,
                           minib_v, minioff_v)
        emit_groups(mc, MAIN)


def kernel(user_ids, item_ids, user_table, item_table, user_bias_table,
           item_bias_table, global_bias):
    gb16 = jnp.broadcast_to(global_bias.astype(jnp.float32), (L,))
    u_rows = _k1(user_ids, user_table.T)
    return _k2(item_ids, item_table.T, u_rows, user_ids,
               user_bias_table.reshape(-1), item_bias_table.reshape(-1),
               gb16)


# speedup vs baseline: 3.3756x; 1.2884x over previous
import functools

import jax
import jax.numpy as jnp
from jax import lax
from jax.experimental import pallas as pl
from jax.experimental.pallas import tpu as pltpu
from jax.experimental.pallas import tpu_sc as plsc

B = 16384
D = 32
L = 16
NC = 2
NS = 16
NW = NC * NS
V = 1000000
G = 512
MAIN = 999936
NCHUNK = MAIN // G
TAIL = V - MAIN

_SC_MESH = plsc.VectorSubcoreMesh(core_axis_name="c", subcore_axis_name="s",
                                  num_cores=NC, num_subcores=NS)
_PARAMS = pltpu.CompilerParams(needs_layout_passes=False)


def _iota():
    return lax.iota(jnp.int32, L)


def _bc(x, dtype=jnp.float32):
    return jnp.full((L,), x, dtype)


def _match_pass(ids_v, wid, matchb_v, matchid_v):
    def body(k, cnt):
        idv = ids_v[pl.ds(k * L, L)]
        bv = k * L + _iota()
        chunk = lax.shift_right_logical(idv, 9)
        m = jnp.logical_and(jnp.bitwise_and(chunk, NW - 1) == wid,
                            idv < MAIN)
        m = jnp.logical_or(
            m, jnp.logical_and(idv >= MAIN, _bc(wid, jnp.int32) == NW - 1))
        pos = cnt + plsc.cumsum(m.astype(jnp.int32)) - 1
        plsc.store_scatter(matchid_v, [pos], idv, mask=m)
        plsc.store_scatter(matchb_v, [pos], bv, mask=m)
        return cnt + plsc.all_reduce_population_count(m)[0]
    return lax.fori_loop(0, B // L, body, 0)


def _chunk_select(lo, width, cnt, matchb_v, matchid_v, minib_v, minioff_v):
    def body(k, mc):
        lane = k * L + _iota()
        idv = matchid_v[pl.ds(k * L, L)]
        bv = matchb_v[pl.ds(k * L, L)]
        m = jnp.logical_and(lane < cnt,
                            jnp.logical_and(idv >= lo, idv < lo + width))
        pos = mc + plsc.cumsum(m.astype(jnp.int32)) - 1
        plsc.store_scatter(minioff_v, [pos], idv - lo, mask=m)
        plsc.store_scatter(minib_v, [pos], bv, mask=m)
        return mc + plsc.all_reduce_population_count(m)[0]
    return lax.fori_loop(0, (cnt + L - 1) // L, body, 0)


@functools.partial(
    pl.kernel,
    out_type=jax.ShapeDtypeStruct((B, 128), jnp.float32),
    mesh=_SC_MESH,
    compiler_params=_PARAMS,
    scratch_types=[
        pltpu.VMEM((B,), jnp.int32),
        pltpu.VMEM((B,), jnp.int32),
        pltpu.VMEM((B,), jnp.int32),
        pltpu.VMEM((B,), jnp.int32),
        pltpu.VMEM((B,), jnp.int32),
        pltpu.VMEM((D, G), jnp.float32),
        pltpu.VMEM((D, G), jnp.float32),
        pltpu.VMEM((L, 128), jnp.float32),
        pltpu.VMEM((L,), jnp.int32),
        pltpu.SemaphoreType.DMA,
        pltpu.SemaphoreType.DMA,
        pltpu.SemaphoreType.DMA,
    ],
)
def _scan(ids_hbm, t_hbm, rows_hbm, ids_v, matchb_v, matchid_v,
          minib_v, minioff_v, slab0_v, slab1_v, rows_v, bidx_v,
          sem0, sem1, sem2):
    wid = lax.axis_index("s") * NC + lax.axis_index("c")
    pltpu.sync_copy(ids_hbm, ids_v)
    cnt = _match_pass(ids_v, wid, matchb_v, matchid_v)

    dlo = _iota()
    dhi = dlo + L
    slabs = (slab0_v, slab1_v)
    sems = (sem0, sem1)

    def issue(c, par):
        @pl.when(c < NCHUNK)
        def _():
            pltpu.async_copy(
                t_hbm.at[:, pl.ds(pl.multiple_of(c * G, 128), G)],
                slabs[par], sems[par])

    def emit_groups(slab_v, mc):
        def gbody(g, carry):
            lane = g * L + _iota()
            valid = lane < mc
            offv = jnp.where(valid, minioff_v[pl.ds(g * L, L)], 0)
            bv = jnp.where(valid, minib_v[pl.ds(g * L, L)], -1)
            bidx_v[...] = bv
            for l in range(L):
                off = _bc(offv[l], jnp.int32)
                rows_v[l, pl.ds(0, L)] = plsc.load_gather(slab_v, [dlo, off])
                rows_v[l, pl.ds(L, L)] = plsc.load_gather(slab_v, [dhi, off])
            pltpu.async_copy(
                rows_v, rows_hbm.at[plsc.Indices(bidx_v, ignored_value=-1)],
                sem2).wait()
            return carry
        lax.fori_loop(0, (mc + L - 1) // L, gbody, 0)

    def process(c, par):
        @pl.when(c < NCHUNK)
        def _():
            slab_v = slabs[par]
            pltpu.make_async_copy(
                t_hbm.at[:, pl.ds(pl.multiple_of(c * G, 128), G)],
                slab_v, sems[par]).wait()
            mc = _chunk_select(c * G, G, cnt, matchb_v, matchid_v,
                               minib_v, minioff_v)
            emit_groups(slab_v, mc)

    issue(wid, 0)

    def pair_body(jp, carry):
        c0 = wid + (jp * 2) * NW
        c1 = wid + (jp * 2 + 1) * NW
        issue(c1, 1)
        process(c0, 0)
        issue(c1 + NW, 0)
        process(c1, 1)
        return carry

    lax.fori_loop(0, (NCHUNK // NW + 2) // 2, pair_body, 0)

    @pl.when(wid == NW - 1)
    def _():
        for dd in range(D):
            pltpu.sync_copy(t_hbm.at[dd, pl.ds(MAIN, TAIL)],
                            slab0_v.at[dd, pl.ds(0, TAIL)])
        mc = _chunk_select(MAIN, TAIL, cnt, matchb_v, matchid_v,
                           minib_v, minioff_v)
        emit_groups(slab0_v, mc)


BPW = B // NW
P = 128


@functools.partial(
    pl.kernel,
    out_type=jax.ShapeDtypeStruct((B,), jnp.float32),
    mesh=_SC_MESH,
    compiler_params=_PARAMS,
    scratch_types=[
        pltpu.VMEM((BPW,), jnp.int32),
        pltpu.VMEM((BPW,), jnp.int32),
        pltpu.VMEM((BPW,), jnp.float32),
        pltpu.VMEM((BPW,), jnp.float32),
        pltpu.VMEM((L,), jnp.float32),
        pltpu.VMEM((P, 128), jnp.float32),
        pltpu.VMEM((P, 128), jnp.float32),
        pltpu.VMEM((BPW,), jnp.float32),
        pltpu.SemaphoreType.DMA,
        pltpu.SemaphoreType.DMA,
        pltpu.SemaphoreType.DMA,
        pltpu.SemaphoreType.DMA,
    ],
)
def _finish(uid_hbm, iid_hbm, urows_hbm, irows_hbm, ub_hbm, ib_hbm, gb_hbm,
            out_hbm, uid_v, iid_v, ubv_v, ibv_v, gb_v, up_v, ip_v, out_v,
            semu, semi, semub, semib):
    wid = lax.axis_index("s") * NC + lax.axis_index("c")
    b0 = wid * BPW
    pltpu.sync_copy(uid_hbm.at[pl.ds(b0, BPW)], uid_v)
    pltpu.sync_copy(iid_hbm.at[pl.ds(b0, BPW)], iid_v)
    pltpu.sync_copy(gb_hbm, gb_v)
    cub = pltpu.async_copy(ub_hbm.at[plsc.Indices(uid_v)], ubv_v, semub)
    cib = pltpu.async_copy(ib_hbm.at[plsc.Indices(iid_v)], ibv_v, semib)
    cub.wait()
    cib.wait()
    gb = gb_v[...]

    def piece(p, carry):
        pb = b0 + p * P
        cu = pltpu.async_copy(urows_hbm.at[pl.ds(pb, P), :], up_v, semu)
        ci = pltpu.async_copy(irows_hbm.at[pl.ds(pb, P), :], ip_v, semi)
        cu.wait()
        ci.wait()

        def group(g, carry2):
            row = g * L + _iota()
            acc = (ubv_v[pl.ds(p * P + g * L, L)]
                   + ibv_v[pl.ds(p * P + g * L, L)] + gb)
            for d in range(D):
                col = _bc(d, jnp.int32)
                u = plsc.load_gather(up_v, [row, col])
                it = plsc.load_gather(ip_v, [row, col])
                acc = acc + u * it
            out_v[pl.ds(p * P + g * L, L)] = acc
            return carry2
        lax.fori_loop(0, P // L, group, 0)
        return carry

    lax.fori_loop(0, BPW // P, piece, 0)
    pltpu.sync_copy(out_v, out_hbm.at[pl.ds(b0, BPW)])


def kernel(user_ids, item_ids, user_table, item_table, user_bias_table,
           item_bias_table, global_bias):
    gb16 = jnp.broadcast_to(global_bias.astype(jnp.float32), (L,))
    u_rows = _scan(user_ids, user_table.T)
    i_rows = _scan(item_ids, item_table.T)
    return _finish(user_ids, item_ids, u_rows, i_rows,
                   user_bias_table.reshape(-1), item_bias_table.reshape(-1),
                   gb16)


# speedup vs baseline: 3.5045x; 1.0382x over previous
import functools

import jax
import jax.numpy as jnp
from jax import lax
from jax.experimental import pallas as pl
from jax.experimental.pallas import tpu as pltpu
from jax.experimental.pallas import tpu_sc as plsc

B = 16384
D = 32
L = 16
NC = 2
NS = 16
NW = NC * NS
V = 1000000
G = 512
MAIN = 999936
NCHUNK = MAIN // G
TAIL = V - MAIN

_SC_MESH = plsc.VectorSubcoreMesh(core_axis_name="c", subcore_axis_name="s",
                                  num_cores=NC, num_subcores=NS)
_PARAMS = pltpu.CompilerParams(needs_layout_passes=False)


def _iota():
    return lax.iota(jnp.int32, L)


def _bc(x, dtype=jnp.float32):
    return jnp.full((L,), x, dtype)


def _match_pass(ids_v, wid, matchb_v, matchid_v):
    def body(k, cnt):
        idv = ids_v[pl.ds(k * L, L)]
        bv = k * L + _iota()
        chunk = lax.shift_right_logical(idv, 9)
        m = jnp.logical_and(jnp.bitwise_and(chunk, NW - 1) == wid,
                            idv < MAIN)
        m = jnp.logical_or(
            m, jnp.logical_and(idv >= MAIN, _bc(wid, jnp.int32) == NW - 1))
        pos = cnt + plsc.cumsum(m.astype(jnp.int32)) - 1
        plsc.store_scatter(matchid_v, [pos], idv, mask=m)
        plsc.store_scatter(matchb_v, [pos], bv, mask=m)
        return cnt + plsc.all_reduce_population_count(m)[0]
    return lax.fori_loop(0, B // L, body, 0)


def _chunk_select(lo, width, cnt, matchb_v, matchid_v, minib_v, minioff_v):
    def body(k, mc):
        lane = k * L + _iota()
        idv = matchid_v[pl.ds(k * L, L)]
        bv = matchb_v[pl.ds(k * L, L)]
        m = jnp.logical_and(lane < cnt,
                            jnp.logical_and(idv >= lo, idv < lo + width))
        pos = mc + plsc.cumsum(m.astype(jnp.int32)) - 1
        plsc.store_scatter(minioff_v, [pos], idv - lo, mask=m)
        plsc.store_scatter(minib_v, [pos], bv, mask=m)
        return mc + plsc.all_reduce_population_count(m)[0]
    return lax.fori_loop(0, (cnt + L - 1) // L, body, 0)


@functools.partial(
    pl.kernel,
    out_type=jax.ShapeDtypeStruct((B, 128), jnp.float32),
    mesh=_SC_MESH,
    compiler_params=_PARAMS,
    scratch_types=[
        pltpu.VMEM((B,), jnp.int32),
        pltpu.VMEM((B,), jnp.int32),
        pltpu.VMEM((B,), jnp.int32),
        pltpu.VMEM((B,), jnp.int32),
        pltpu.VMEM((B,), jnp.int32),
        pltpu.VMEM((D, G), jnp.float32),
        pltpu.VMEM((D, G), jnp.float32),
        pltpu.VMEM((L, 128), jnp.float32),
        pltpu.VMEM((L,), jnp.int32),
        pltpu.SemaphoreType.DMA,
        pltpu.SemaphoreType.DMA,
        pltpu.SemaphoreType.DMA,
        pltpu.SemaphoreType.DMA,
        pltpu.SemaphoreType.DMA,
        pltpu.SemaphoreType.DMA,
        pltpu.SemaphoreType.DMA,
        pltpu.SemaphoreType.DMA,
        pltpu.SemaphoreType.DMA,
    ],
)
def _scan(ids_hbm, t_hbm, rows_hbm, ids_v, matchb_v, matchid_v,
          minib_v, minioff_v, slab0_v, slab1_v, rows_v, bidx_v,
          s00, s01, s02, s03, s10, s11, s12, s13, sem2):
    wid = lax.axis_index("s") * NC + lax.axis_index("c")
    pltpu.sync_copy(ids_hbm, ids_v)
    cnt = _match_pass(ids_v, wid, matchb_v, matchid_v)

    dlo = _iota()
    dhi = dlo + L
    slabs = (slab0_v, slab1_v)
    sems = ((s00, s01, s02, s03), (s10, s11, s12, s13))

    def issue(c, par):
        @pl.when(c < NCHUNK)
        def _():
            for q in range(4):
                pltpu.async_copy(
                    t_hbm.at[pl.ds(q * 8, 8),
                             pl.ds(pl.multiple_of(c * G, 128), G)],
                    slabs[par].at[pl.ds(q * 8, 8), :], sems[par][q])

    def emit_groups(slab_v, mc):
        def gbody(g, carry):
            lane = g * L + _iota()
            valid = lane < mc
            offv = jnp.where(valid, minioff_v[pl.ds(g * L, L)], 0)
            bv = jnp.where(valid, minib_v[pl.ds(g * L, L)], -1)
            bidx_v[...] = bv
            for l in range(L):
                off = _bc(offv[l], jnp.int32)
                rows_v[l, pl.ds(0, L)] = plsc.load_gather(slab_v, [dlo, off])
                rows_v[l, pl.ds(L, L)] = plsc.load_gather(slab_v, [dhi, off])
            pltpu.async_copy(
                rows_v, rows_hbm.at[plsc.Indices(bidx_v, ignored_value=-1)],
                sem2).wait()
            return carry
        lax.fori_loop(0, (mc + L - 1) // L, gbody, 0)

    def process(c, par):
        @pl.when(c < NCHUNK)
        def _():
            slab_v = slabs[par]
            for q in range(4):
                pltpu.make_async_copy(
                    t_hbm.at[pl.ds(q * 8, 8),
                             pl.ds(pl.multiple_of(c * G, 128), G)],
                    slab_v.at[pl.ds(q * 8, 8), :], sems[par][q]).wait()
            mc = _chunk_select(c * G, G, cnt, matchb_v, matchid_v,
                               minib_v, minioff_v)
            emit_groups(slab_v, mc)

    issue(wid, 0)

    def pair_body(jp, carry):
        c0 = wid + (jp * 2) * NW
        c1 = wid + (jp * 2 + 1) * NW
        issue(c1, 1)
        process(c0, 0)
        issue(c1 + NW, 0)
        process(c1, 1)
        return carry

    lax.fori_loop(0, (NCHUNK // NW + 2) // 2, pair_body, 0)

    @pl.when(wid == NW - 1)
    def _():
        for dd in range(D):
            pltpu.sync_copy(t_hbm.at[dd, pl.ds(MAIN, TAIL)],
                            slab0_v.at[dd, pl.ds(0, TAIL)])
        mc = _chunk_select(MAIN, TAIL, cnt, matchb_v, matchid_v,
                           minib_v, minioff_v)
        emit_groups(slab0_v, mc)


BPW = B // NW
P = 128


@functools.partial(
    pl.kernel,
    out_type=jax.ShapeDtypeStruct((B,), jnp.float32),
    mesh=_SC_MESH,
    compiler_params=_PARAMS,
    scratch_types=[
        pltpu.VMEM((BPW,), jnp.int32),
        pltpu.VMEM((BPW,), jnp.int32),
        pltpu.VMEM((BPW,), jnp.float32),
        pltpu.VMEM((BPW,), jnp.float32),
        pltpu.VMEM((L,), jnp.float32),
        pltpu.VMEM((P, 128), jnp.float32),
        pltpu.VMEM((P, 128), jnp.float32),
        pltpu.VMEM((BPW,), jnp.float32),
        pltpu.SemaphoreType.DMA,
        pltpu.SemaphoreType.DMA,
        pltpu.SemaphoreType.DMA,
        pltpu.SemaphoreType.DMA,
    ],
)
def _finish(uid_hbm, iid_hbm, urows_hbm, irows_hbm, ub_hbm, ib_hbm, gb_hbm,
            out_hbm, uid_v, iid_v, ubv_v, ibv_v, gb_v, up_v, ip_v, out_v,
            semu, semi, semub, semib):
    wid = lax.axis_index("s") * NC + lax.axis_index("c")
    b0 = wid * BPW
    pltpu.sync_copy(uid_hbm.at[pl.ds(b0, BPW)], uid_v)
    pltpu.sync_copy(iid_hbm.at[pl.ds(b0, BPW)], iid_v)
    pltpu.sync_copy(gb_hbm, gb_v)
    cub = pltpu.async_copy(ub_hbm.at[plsc.Indices(uid_v)], ubv_v, semub)
    cib = pltpu.async_copy(ib_hbm.at[plsc.Indices(iid_v)], ibv_v, semib)
    cub.wait()
    cib.wait()
    gb = gb_v[...]

    def piece(p, carry):
        pb = b0 + p * P
        cu = pltpu.async_copy(urows_hbm.at[pl.ds(pb, P), :], up_v, semu)
        ci = pltpu.async_copy(irows_hbm.at[pl.ds(pb, P), :], ip_v, semi)
        cu.wait()
        ci.wait()

        def group(g, carry2):
            row = g * L + _iota()
            acc = (ubv_v[pl.ds(p * P + g * L, L)]
                   + ibv_v[pl.ds(p * P + g * L, L)] + gb)
            for d in range(D):
                col = _bc(d, jnp.int32)
                u = plsc.load_gather(up_v, [row, col])
                it = plsc.load_gather(ip_v, [row, col])
                acc = acc + u * it
            out_v[pl.ds(p * P + g * L, L)] = acc
            return carry2
        lax.fori_loop(0, P // L, group, 0)
        return carry

    lax.fori_loop(0, BPW // P, piece, 0)
    pltpu.sync_copy(out_v, out_hbm.at[pl.ds(b0, BPW)])


def kernel(user_ids, item_ids, user_table, item_table, user_bias_table,
           item_bias_table, global_bias):
    gb16 = jnp.broadcast_to(global_bias.astype(jnp.float32), (L,))
    u_rows = _scan(user_ids, user_table.T)
    i_rows = _scan(item_ids, item_table.T)
    return _finish(user_ids, item_ids, u_rows, i_rows,
                   user_bias_table.reshape(-1), item_bias_table.reshape(-1),
                   gb16)


# speedup vs baseline: 3.6868x; 1.0520x over previous
import functools

import jax
import jax.numpy as jnp
from jax import lax
from jax.experimental import pallas as pl
from jax.experimental.pallas import tpu as pltpu
from jax.experimental.pallas import tpu_sc as plsc

B = 16384
D = 32
L = 16
NC = 2
NS = 16
NW = NC * NS
V = 1000000
G = 512
MAIN = 999936
NCHUNK = MAIN // G
TAIL = V - MAIN
NJ = 62
JT = NJ
JX = NJ + 1
CAP = B + 64 * L

_SC_MESH = plsc.VectorSubcoreMesh(core_axis_name="c", subcore_axis_name="s",
                                  num_cores=NC, num_subcores=NS)
_PARAMS = pltpu.CompilerParams(needs_layout_passes=False)


def _iota():
    return lax.iota(jnp.int32, L)


def _bc(x, dtype=jnp.float32):
    return jnp.full((L,), x, dtype)


def _match_pass(ids_v, wid, matchb_v, matchid_v):
    def body(k, cnt):
        idv = ids_v[pl.ds(k * L, L)]
        bv = k * L + _iota()
        chunk = lax.shift_right_logical(idv, 9)
        m = jnp.logical_and(jnp.bitwise_and(chunk, NW - 1) == wid,
                            idv < MAIN)
        m = jnp.logical_or(
            m, jnp.logical_and(idv >= MAIN, _bc(wid, jnp.int32) == NW - 1))
        pos = cnt + plsc.cumsum(m.astype(jnp.int32)) - 1
        plsc.store_scatter(matchid_v, [pos], idv, mask=m)
        plsc.store_scatter(matchb_v, [pos], bv, mask=m)
        return cnt + plsc.all_reduce_population_count(m)[0]
    return lax.fori_loop(0, B // L, body, 0)


def _bucket_j(idv, wid, cnt, lane):
    chunk = lax.shift_right_logical(idv, 9)
    j = lax.shift_right_logical(chunk - wid, 5)
    j = jnp.where(idv >= MAIN, JT, j)
    return jnp.where(lane < cnt, j, JX)


def _bucket_sort(wid, cnt, matchb_v, matchid_v, minib_v, minioff_v,
                 hist_v, starts_v):
    ones = _bc(1, jnp.int32)
    zeros = jnp.zeros((L,), jnp.int32)
    nloops = (cnt + L - 1) // L
    for k in range(4):
        hist_v[pl.ds(k * L, L)] = zeros

    def hbody(k, carry):
        lane = k * L + _iota()
        idv = matchid_v[pl.ds(k * L, L)]
        jv = _bucket_j(idv, wid, cnt, lane)
        for l in range(L):
            plsc.addupdate_scatter(hist_v, [_bc(jv[l], jnp.int32)], ones,
                                   mask=_iota() == l)
        return carry
    lax.fori_loop(0, nloops, hbody, 0)

    def pbody(k, carry):
        h = hist_v[pl.ds(k * L, L)]
        sal = jnp.bitwise_and(h + (L - 1), _bc(~(L - 1), jnp.int32))
        inc = plsc.cumsum(sal)
        starts_v[pl.ds(k * L, L)] = carry + inc - sal
        return carry + inc[L - 1]
    total = lax.fori_loop(0, 4, pbody, 0)

    neg = _bc(-1, jnp.int32)

    def cbody(k, carry):
        minib_v[pl.ds(k * L, L)] = neg
        return carry
    lax.fori_loop(0, (total + L - 1) // L, cbody, 0)

    def rbody(k, carry):
        hist_v[pl.ds(k * L, L)] = starts_v[pl.ds(k * L, L)]
        return carry
    lax.fori_loop(0, 4, rbody, 0)

    def sbody(k, carry):
        lane = k * L + _iota()
        idv = matchid_v[pl.ds(k * L, L)]
        bv = matchb_v[pl.ds(k * L, L)]
        jv = _bucket_j(idv, wid, cnt, lane)
        for l in range(L):
            lm = _iota() == l
            jl = _bc(jv[l], jnp.int32)
            posl = _bc(plsc.load_gather(hist_v, [jl])[0], jnp.int32)
            plsc.store_scatter(minioff_v, [posl], idv, mask=lm)
            plsc.store_scatter(minib_v, [posl], bv, mask=lm)
            plsc.addupdate_scatter(hist_v, [jl], ones, mask=lm)
        return carry
    lax.fori_loop(0, nloops, sbody, 0)


@functools.partial(
    pl.kernel,
    out_type=jax.ShapeDtypeStruct((B, 128), jnp.float32),
    mesh=_SC_MESH,
    compiler_params=_PARAMS,
    scratch_types=[
        pltpu.VMEM((B,), jnp.int32),
        pltpu.VMEM((B,), jnp.int32),
        pltpu.VMEM((B,), jnp.int32),
        pltpu.VMEM((CAP,), jnp.int32),
        pltpu.VMEM((CAP,), jnp.int32),
        pltpu.VMEM((64,), jnp.int32),
        pltpu.VMEM((64,), jnp.int32),
        pltpu.VMEM((D, G), jnp.float32),
        pltpu.VMEM((D, G), jnp.float32),
        pltpu.VMEM((L, 128), jnp.float32),
        pltpu.VMEM((L,), jnp.int32),
        pltpu.SemaphoreType.DMA,
        pltpu.SemaphoreType.DMA,
        pltpu.SemaphoreType.DMA,
        pltpu.SemaphoreType.DMA,
        pltpu.SemaphoreType.DMA,
        pltpu.SemaphoreType.DMA,
        pltpu.SemaphoreType.DMA,
        pltpu.SemaphoreType.DMA,
        pltpu.SemaphoreType.DMA,
    ],
)
def _scan(ids_hbm, t_hbm, rows_hbm, ids_v, matchb_v, matchid_v,
          minib_v, minioff_v, hist_v, starts_v, slab0_v, slab1_v,
          rows_v, bidx_v, s00, s01, s02, s03, s10, s11, s12, s13, sem2):
    wid = lax.axis_index("s") * NC + lax.axis_index("c")
    pltpu.sync_copy(ids_hbm, ids_v)
    cnt = _match_pass(ids_v, wid, matchb_v, matchid_v)
    _bucket_sort(wid, cnt, matchb_v, matchid_v, minib_v, minioff_v,
                 hist_v, starts_v)

    dlo = _iota()
    dhi = dlo + L
    slabs = (slab0_v, slab1_v)
    sems = ((s00, s01, s02, s03), (s10, s11, s12, s13))

    def issue(c, par):
        @pl.when(c < NCHUNK)
        def _():
            for q in range(4):
                pltpu.async_copy(
                    t_hbm.at[pl.ds(q * 8, 8),
                             pl.ds(pl.multiple_of(c * G, 128), G)],
                    slabs[par].at[pl.ds(q * 8, 8), :], sems[par][q])

    def emit_groups(slab_v, j, lo):
        gstart = plsc.load_gather(starts_v, [_bc(j, jnp.int32)])[0]
        gend = plsc.load_gather(starts_v, [_bc(j + 1, jnp.int32)])[0]

        def gbody(g, carry):
            sl = pl.ds(gstart + g * L, L)
            bv = minib_v[sl]
            idvb = minioff_v[sl]
            valid = bv >= 0
            offv = jnp.where(valid, idvb - lo, 0)
            bidx_v[...] = jnp.where(valid, bv, -1)
            for l in range(L):
                off = _bc(offv[l], jnp.int32)
                rows_v[l, pl.ds(0, L)] = plsc.load_gather(slab_v, [dlo, off])
                rows_v[l, pl.ds(L, L)] = plsc.load_gather(slab_v, [dhi, off])
            pltpu.async_copy(
                rows_v, rows_hbm.at[plsc.Indices(bidx_v, ignored_value=-1)],
                sem2).wait()
            return carry
        lax.fori_loop(0, lax.shift_right_logical(gend - gstart, 4), gbody, 0)

    def process(c, j, par):
        @pl.when(c < NCHUNK)
        def _():
            slab_v = slabs[par]
            for q in range(4):
                pltpu.make_async_copy(
                    t_hbm.at[pl.ds(q * 8, 8),
                             pl.ds(pl.multiple_of(c * G, 128), G)],
                    slab_v.at[pl.ds(q * 8, 8), :], sems[par][q]).wait()
            emit_groups(slab_v, j, c * G)

    issue(wid, 0)

    def pair_body(jp, carry):
        j0 = jp * 2
        j1 = jp * 2 + 1
        c0 = wid + j0 * NW
        c1 = wid + j1 * NW
        issue(c1, 1)
        process(c0, j0, 0)
        issue(c1 + NW, 0)
        process(c1, j1, 1)
        return carry

    lax.fori_loop(0, (NCHUNK // NW + 2) // 2, pair_body, 0)

    @pl.when(wid == NW - 1)
    def _():
        for dd in range(D):
            pltpu.sync_copy(t_hbm.at[dd, pl.ds(MAIN, TAIL)],
                            slab0_v.at[dd, pl.ds(0, TAIL)])
        emit_groups(slab0_v, JT, MAIN)


BPW = B // NW
P = 128


@functools.partial(
    pl.kernel,
    out_type=jax.ShapeDtypeStruct((B,), jnp.float32),
    mesh=_SC_MESH,
    compiler_params=_PARAMS,
    scratch_types=[
        pltpu.VMEM((BPW,), jnp.int32),
        pltpu.VMEM((BPW,), jnp.int32),
        pltpu.VMEM((BPW,), jnp.float32),
        pltpu.VMEM((BPW,), jnp.float32),
        pltpu.VMEM((L,), jnp.float32),
        pltpu.VMEM((P, 128), jnp.float32),
        pltpu.VMEM((P, 128), jnp.float32),
        pltpu.VMEM((BPW,), jnp.float32),
        pltpu.SemaphoreType.DMA,
        pltpu.SemaphoreType.DMA,
        pltpu.SemaphoreType.DMA,
        pltpu.SemaphoreType.DMA,
    ],
)
def _finish(uid_hbm, iid_hbm, urows_hbm, irows_hbm, ub_hbm, ib_hbm, gb_hbm,
            out_hbm, uid_v, iid_v, ubv_v, ibv_v, gb_v, up_v, ip_v, out_v,
            semu, semi, semub, semib):
    wid = lax.axis_index("s") * NC + lax.axis_index("c")
    b0 = wid * BPW
    pltpu.sync_copy(uid_hbm.at[pl.ds(b0, BPW)], uid_v)
    pltpu.sync_copy(iid_hbm.at[pl.ds(b0, BPW)], iid_v)
    pltpu.sync_copy(gb_hbm, gb_v)
    cub = pltpu.async_copy(ub_hbm.at[plsc.Indices(uid_v)], ubv_v, semub)
    cib = pltpu.async_copy(ib_hbm.at[plsc.Indices(iid_v)], ibv_v, semib)
    cub.wait()
    cib.wait()
    gb = gb_v[...]

    def piece(p, carry):
        pb = b0 + p * P
        cu = pltpu.async_copy(urows_hbm.at[pl.ds(pb, P), :], up_v, semu)
        ci = pltpu.async_copy(irows_hbm.at[pl.ds(pb, P), :], ip_v, semi)
        cu.wait()
        ci.wait()

        def group(g, carry2):
            row = g * L + _iota()
            acc = (ubv_v[pl.ds(p * P + g * L, L)]
                   + ibv_v[pl.ds(p * P + g * L, L)] + gb)
            for d in range(D):
                col = _bc(d, jnp.int32)
                u = plsc.load_gather(up_v, [row, col])
                it = plsc.load_gather(ip_v, [row, col])
                acc = acc + u * it
            out_v[pl.ds(p * P + g * L, L)] = acc
            return carry2
        lax.fori_loop(0, P // L, group, 0)
        return carry

    lax.fori_loop(0, BPW // P, piece, 0)
    pltpu.sync_copy(out_v, out_hbm.at[pl.ds(b0, BPW)])


def kernel(user_ids, item_ids, user_table, item_table, user_bias_table,
           item_bias_table, global_bias):
    gb16 = jnp.broadcast_to(global_bias.astype(jnp.float32), (L,))
    u_rows = _scan(user_ids, user_table.T)
    i_rows = _scan(item_ids, item_table.T)
    return _finish(user_ids, item_ids, u_rows, i_rows,
                   user_bias_table.reshape(-1), item_bias_table.reshape(-1),
                   gb16)
